# baseline (speedup 1.0000x reference)
import functools

import jax
import jax.numpy as jnp
from jax.experimental import pallas as pl
from jax.experimental.pallas import tpu as pltpu

_K = 5
_TOPN = _K + 1
_DEPTH = _K + 2
_NEG = float("-inf")


def _extract_top(cat, n):
    cat_cols = jax.lax.broadcasted_iota(jnp.int32, cat.shape, 1)
    big = jnp.int32(2**31 - 1)
    outs = []
    for _ in range(n):
        v = jnp.max(cat, axis=1, keepdims=True)
        hit = cat == v
        idx = jnp.min(jnp.where(hit, cat_cols, big), axis=1, keepdims=True)
        cat = jnp.where(cat_cols == idx, _NEG, cat)
        outs.append(v)
    return jnp.concatenate(outs, axis=1)


_SORT8_NET = (
    (0, 1), (2, 3), (4, 5), (6, 7),
    (0, 2), (1, 3), (4, 6), (5, 7),
    (1, 2), (5, 6),
    (0, 4), (1, 5), (2, 6), (3, 7),
    (2, 4), (3, 5),
    (1, 2), (3, 4), (5, 6),
)


def _sort8(vs):
    vs = list(vs)
    for a, b in _SORT8_NET:
        hi = jnp.maximum(vs[a], vs[b])
        lo = jnp.minimum(vs[a], vs[b])
        vs[a], vs[b] = hi, lo
    return vs


def _merge8(tup, s):
    m = [jnp.maximum(s[i], tup[7 - i]) for i in range(8)]
    for dist in (4, 2, 1):
        for i in range(8):
            if (i // dist) % 2 == 0:
                hi = jnp.maximum(m[i], m[i + dist])
                lo = jnp.minimum(m[i], m[i + dist])
                m[i], m[i + dist] = hi, lo
    return m


def _body(yhi_ref, ylo_ref, x_ref, out_ref, *, rows, nf, w, total_rows):
    i = pl.program_id(0)
    ylo = ylo_ref[...]

    unroll = 16
    def step(j, carry):
        ta = list(carry[:8])
        tb = list(carry[8:])
        for c in range(0, unroll, 16):
            base = unroll * j + c
            batcha = [x_ref[base + q, :, :] for q in range(8)]
            batchb = [x_ref[base + 8 + q, :, :] for q in range(8)]
            ta = _merge8(ta, _sort8(batcha))
            tb = _merge8(tb, _sort8(batchb))
        return (*ta, *tb)

    init = tuple(jnp.full((rows, w), _NEG, jnp.float32) for _ in range(16))
    res = jax.lax.fori_loop(0, nf // unroll, step, init)

    cand = jnp.concatenate(res, axis=1)
    top7 = _extract_top(cand, _DEPTH)

    lane = jax.lax.broadcasted_iota(jnp.int32, (rows, w), 1)
    tmat = jnp.concatenate(
        [x_ref[yhi_ref[r, 0], r, :].reshape(1, w) for r in range(rows)],
        axis=0)
    t = jnp.sum(jnp.where(lane == ylo, tmat, 0.0), axis=1, keepdims=True)

    m = top7[:, :1]
    m3 = m.reshape(1, rows, 1)
    xb = x_ref[...]
    s = jnp.sum(jnp.sum(jnp.exp(xb - m3), axis=2), axis=0).reshape(rows, 1)

    l6 = top7[:, _TOPN:]
    dropval = jnp.where(t >= l6, t, l6)
    cols7 = jax.lax.broadcasted_iota(jnp.int32, (rows, _DEPTH), 1)
    hit = top7 == dropval
    dropidx = jnp.min(jnp.where(hit, cols7, jnp.int32(2**31 - 1)),
                      axis=1, keepdims=True)
    keep = cols7 != dropidx

    pp = jnp.exp(t - m) / s
    pn = jnp.exp(top7 - m) / s
    terms = (1.0 + pn - pp) ** 2
    loss = jnp.sum(jnp.where(keep, terms, 0.0), axis=1, keepdims=True) / _K
    part = (jnp.sum(loss) / total_rows).reshape(1, 1)

    @pl.when(i == 0)
    def _init_out():
        out_ref[...] = jnp.zeros((1, 1), jnp.float32)

    out_ref[...] += part


@functools.partial(jax.jit, static_argnames=("w", "rblk"))
def _run(pred, y2, w, rblk):
    rows, nclass = pred.shape
    nf = nclass // w
    pred3 = jnp.transpose(pred.reshape(rows, nf, w), (1, 0, 2))
    yhi = y2 // w
    ylo = y2 % w
    body = functools.partial(_body, rows=rblk, nf=nf, w=w, total_rows=rows)
    out = pl.pallas_call(
        body,
        grid=(rows // rblk,),
        in_specs=[
            pl.BlockSpec((rblk, 1), lambda i: (i, 0),
                         memory_space=pltpu.SMEM),
            pl.BlockSpec((rblk, 1), lambda i: (i, 0)),
            pl.BlockSpec((nf, rblk, w), lambda i: (0, i, 0)),
        ],
        out_specs=pl.BlockSpec((1, 1), lambda i: (0, 0)),
        out_shape=jax.ShapeDtypeStruct((1, 1), jnp.float32),
        compiler_params=pltpu.CompilerParams(
            dimension_semantics=("arbitrary",)),
    )(yhi, ylo, pred3)
    return out[0, 0]


def kernel(pred, y, epoch=0):
    rows, nclass = pred.shape
    for cand in (250, 125, 100, 50, 25, 20, 10, 5, 4, 2, 1):
        if nclass % (cand * 16) == 0:
            w = cand
            break
    else:
        w = nclass
    rblk = 8 if rows % 8 == 0 else rows
    y2 = y.reshape(rows, 1).astype(jnp.int32)
    return _run(pred, y2, w, rblk)
